# baseline (speedup 1.0000x reference)
import functools

import jax
import jax.numpy as jnp
from jax import lax
from jax.experimental import pallas as pl
from jax.experimental.pallas import tpu as pltpu
from jax.experimental.pallas import tpu_sc as plsc

B = 16384
HL = 50
D = 32
DB = 16
OUT_D = 224
NC, NS = 2, 16
NW = NC * NS
BPW = B // NW
CH = 128
NCH = BPW // CH

COL_NUM, COL_C0, COL_C1, COL_C2, COL_C3 = 0, 32, 64, 96, 128
COL_B0, COL_B1, COL_H = 160, 176, 192

_mesh = plsc.VectorSubcoreMesh(core_axis_name="c", subcore_axis_name="s")


def _body(num_h, c0_h, c1_h, c2_h, c3_h, b0_h, b1_h, histT_h, wt_h, b_h,
          Ec0, Ec1, Ec2, Ec3, Eb0, Eb1, Eh,
          o_num, o_c0, o_c1, o_c2, o_c3, o_b0, o_b1, o_h,
          idx0, idx1, idx2, idx3, ib0, ib1, histT_v, num_v, wt_v, b_v,
          bufA, bufB, bufA_bf, bb0, bb1, acc, num_out, inv_v,
          sin, s0, s1, s2, sout, sh0, sh1):
    cid = lax.axis_index("c")
    sid = lax.axis_index("s")
    wid = sid * NC + cid
    base = wid * BPW
    qbase = wid * NCH

    ins = [
        pltpu.async_copy(c0_h.at[pl.ds(qbase, NCH)], idx0, sin),
        pltpu.async_copy(c1_h.at[pl.ds(qbase, NCH)], idx1, sin),
        pltpu.async_copy(c2_h.at[pl.ds(qbase, NCH)], idx2, sin),
        pltpu.async_copy(c3_h.at[pl.ds(qbase, NCH)], idx3, sin),
        pltpu.async_copy(b0_h.at[pl.ds(qbase, NCH)], ib0, sin),
        pltpu.async_copy(b1_h.at[pl.ds(qbase, NCH)], ib1, sin),
        pltpu.async_copy(histT_h.at[:, pl.ds(qbase, NCH), :], histT_v, sin),
        pltpu.async_copy(num_h.at[pl.ds(base, BPW)], num_v, sin),
        pltpu.async_copy(wt_h, wt_v, sin),
        pltpu.async_copy(b_h, b_v, sin),
    ]
    for cp in ins:
        cp.wait()

    def gather(table, idx, dst, sem):
        ds = [pltpu.async_copy(table.at[idx.at[q]],
                               dst.at[pl.ds(q * CH, CH), :], sem)
              for q in range(NCH)]
        return ds

    def wait_all(ds):
        for d in ds:
            d.wait()

    g0 = gather(Ec0, idx0, bufA_bf, s0)
    g1 = gather(Ec1, idx1, bufB, s1)
    g2 = gather(Ec2, idx2, bufA, sh0)
    gb0 = gather(Eb0, ib0, bb0, s2)
    gb1 = gather(Eb1, ib1, bb1, s2)
    wait_all(g0)
    two = jnp.full((16,), 2, jnp.int32)
    cole = lax.iota(jnp.int32, 16) * two
    colo = cole + jnp.full((16,), 1, jnp.int32)

    def cvt_body(r, carry):
        v = bufA_bf[r]
        ae, ao = plsc.unpack(v, format=plsc.PackFormat.INTERLEAVED)
        rows = jnp.broadcast_to(r, (16,))
        plsc.store_scatter(num_out, [rows, cole], ae)
        plsc.store_scatter(num_out, [rows, colo], ao)
        return carry

    lax.fori_loop(0, BPW, cvt_body, 0, unroll=4)
    w0 = pltpu.async_copy(num_out, o_c0.at[pl.ds(base, BPW)], sout)
    wait_all(g1)
    w1 = pltpu.async_copy(bufB, o_c1.at[pl.ds(base, BPW)], sout)
    wait_all(g2)
    w2 = pltpu.async_copy(bufA, o_c2.at[pl.ds(base, BPW)], sout)
    w1.wait()
    g3 = gather(Ec3, idx3, bufB, s1)
    wait_all(gb0)
    wb0 = pltpu.async_copy(bb0, o_b0.at[pl.ds(base, BPW)], sout)
    wait_all(gb1)
    wb1 = pltpu.async_copy(bb1, o_b1.at[pl.ds(base, BPW)], sout)
    wait_all(g3)
    w3 = pltpu.async_copy(bufB, o_c3.at[pl.ds(base, BPW)], sout)
    w0.wait()

    wvec = [(wt_v[k, pl.ds(0, 16)], wt_v[k, pl.ds(16, 16)]) for k in range(13)]
    bv0 = b_v[pl.ds(0, 16)]
    bv1 = b_v[pl.ds(16, 16)]

    def num_body(r, carry):
        nv = num_v[r, pl.ds(0, 16)]
        a0, a1 = bv0, bv1
        for k in range(13):
            sv = jnp.broadcast_to(nv[k], (16,))
            a0 = a0 + sv * wvec[k][0]
            a1 = a1 + sv * wvec[k][1]
        num_out[r, pl.ds(0, 16)] = a0
        num_out[r, pl.ds(16, 16)] = a1
        return carry

    lax.fori_loop(0, BPW, num_body, 0, unroll=2)
    wn = pltpu.async_copy(num_out, o_num.at[pl.ds(base, BPW)], sout)

    w2.wait()
    w3.wait()
    wn.wait()
    gbuf = (bufA, bufB, num_out)
    shs = (sh0, sh1, s0)

    def hist_gather(j, p):
        return [pltpu.async_copy(Eh.at[histT_v.at[j, q]],
                                 gbuf[p].at[pl.ds(q * CH, CH), :], shs[p])
                for q in range(NCH)]

    def mk_acc_loop(gb, first):
        def acc_body(r, carry):
            g0v = gb[r, pl.ds(0, 16)]
            g1v = gb[r, pl.ds(16, 16)]
            if first:
                acc[r, pl.ds(0, 16)] = g0v
                acc[r, pl.ds(16, 16)] = g1v
            else:
                plsc.addupdate(acc.at[r, pl.ds(0, 16)], g0v)
                plsc.addupdate(acc.at[r, pl.ds(16, 16)], g1v)
            return carry
        return acc_body

    pend = [hist_gather(0, 0), hist_gather(1, 1), hist_gather(2, 2)]
    for j in range(HL):
        p = j % 3
        wait_all(pend[p])
        lax.fori_loop(0, BPW, mk_acc_loop(gbuf[p], j == 0), 0, unroll=8)
        if j + 3 < HL:
            pend[p] = hist_gather(j + 3, p)

    def cnt_body(g, carry):
        q = g // (CH // 16)
        off = (g % (CH // 16)) * 16
        c = jnp.zeros((16,), jnp.float32)
        zi = jnp.zeros((16,), jnp.int32)
        one = jnp.full((16,), 1.0, jnp.float32)
        zf = jnp.zeros((16,), jnp.float32)
        for j in range(HL):
            iv = histT_v[j, q, pl.ds(off, 16)]
            c = c + jnp.where(iv != zi, one, zf)
        inv_v[pl.ds(g * 16, 16)] = one / jnp.maximum(c, jnp.full((16,), 1e-6, jnp.float32))
        return carry

    lax.fori_loop(0, BPW // 16, cnt_body, 0)

    def scale_body(g, carry):
        iv = inv_v[pl.ds(g * 16, 16)]
        for i in range(16):
            sv = jnp.broadcast_to(iv[i], (16,))
            r = g * 16 + i
            acc[r, pl.ds(0, 16)] = acc[r, pl.ds(0, 16)] * sv
            acc[r, pl.ds(16, 16)] = acc[r, pl.ds(16, 16)] * sv
        return carry

    lax.fori_loop(0, BPW // 16, scale_body, 0)
    wh = pltpu.async_copy(acc, o_h.at[pl.ds(base, BPW)], sout)

    for d in (wb0, wb1, wh):
        d.wait()


_encode = pl.kernel(
    _body,
    out_type=[
        jax.ShapeDtypeStruct((B, D), jnp.float32),
        jax.ShapeDtypeStruct((B, D), jnp.float32),
        jax.ShapeDtypeStruct((B, D), jnp.float32),
        jax.ShapeDtypeStruct((B, D), jnp.float32),
        jax.ShapeDtypeStruct((B, D), jnp.float32),
        jax.ShapeDtypeStruct((B, DB), jnp.float32),
        jax.ShapeDtypeStruct((B, DB), jnp.float32),
        jax.ShapeDtypeStruct((B, D), jnp.float32),
    ],
    mesh=_mesh,
    compiler_params=pltpu.CompilerParams(use_tc_tiling_on_sc=False,
                                         needs_layout_passes=False),
    scratch_types=[
        pltpu.VMEM((NCH, CH), jnp.int32),
        pltpu.VMEM((NCH, CH), jnp.int32),
        pltpu.VMEM((NCH, CH), jnp.int32),
        pltpu.VMEM((NCH, CH), jnp.int32),
        pltpu.VMEM((NCH, CH), jnp.int32),
        pltpu.VMEM((NCH, CH), jnp.int32),
        pltpu.VMEM((HL, NCH, CH), jnp.int32),
        pltpu.VMEM((BPW, 16), jnp.float32),
        pltpu.VMEM((13, D), jnp.float32),
        pltpu.VMEM((D,), jnp.float32),
        pltpu.VMEM((BPW, D), jnp.float32),
        pltpu.VMEM((BPW, D), jnp.float32),
        pltpu.VMEM((BPW, D), jnp.bfloat16),
        pltpu.VMEM((BPW, DB), jnp.float32),
        pltpu.VMEM((BPW, DB), jnp.float32),
        pltpu.VMEM((BPW, D), jnp.float32),
        pltpu.VMEM((BPW, D), jnp.float32),
        pltpu.VMEM((BPW,), jnp.float32),
        pltpu.SemaphoreType.DMA,
        pltpu.SemaphoreType.DMA,
        pltpu.SemaphoreType.DMA,
        pltpu.SemaphoreType.DMA,
        pltpu.SemaphoreType.DMA,
        pltpu.SemaphoreType.DMA,
        pltpu.SemaphoreType.DMA,
    ],
)


def kernel(numeric, cat_0, cat_1, cat_2, cat_3, bkt_0, bkt_1, hist_items,
           W_num, b_num, E_cat_0, E_cat_1, E_cat_2, E_cat_3,
           E_bkt_0, E_bkt_1, E_hist):
    numeric = jnp.pad(numeric, ((0, 0), (0, 3)))
    hist_T = jnp.transpose(hist_items).reshape(HL, B // CH, CH)
    c0 = cat_0.reshape(B // CH, CH)
    c1 = cat_1.reshape(B // CH, CH)
    c2 = cat_2.reshape(B // CH, CH)
    c3 = cat_3.reshape(B // CH, CH)
    b0 = bkt_0.reshape(B // CH, CH)
    b1 = bkt_1.reshape(B // CH, CH)
    w_t = jnp.transpose(W_num)
    outs = _encode(numeric, c0, c1, c2, c3, b0, b1, hist_T, w_t, b_num,
                   E_cat_0.astype(jnp.bfloat16), E_cat_1, E_cat_2, E_cat_3,
                   E_bkt_0, E_bkt_1, E_hist)
    return jnp.concatenate(outs, axis=-1)


# speedup vs baseline: 1.1862x; 1.1862x over previous
import functools

import jax
import jax.numpy as jnp
from jax import lax
from jax.experimental import pallas as pl
from jax.experimental.pallas import tpu as pltpu
from jax.experimental.pallas import tpu_sc as plsc

B = 16384
HL = 50
D = 32
DB = 16
OUT_D = 224
NC, NS = 2, 16
NW = NC * NS
BPW = B // NW
CH = 128
NCH = BPW // CH

COL_NUM, COL_C0, COL_C1, COL_C2, COL_C3 = 0, 32, 64, 96, 128
COL_B0, COL_B1, COL_H = 160, 176, 192

_mesh = plsc.VectorSubcoreMesh(core_axis_name="c", subcore_axis_name="s")


def _body(num_h, c0_h, c1_h, c2_h, c3_h, b0_h, b1_h, histT_h, wt_h, b_h,
          Ec0, Ec1, Ec2, Ec3, Eb0, Eb1, Eh,
          o_num, o_c0, o_c1, o_c2, o_c3, o_b0, o_b1, o_h,
          idx0, idx1, idx2, idx3, ib0, ib1, histT_v, num_v, wt_v, b_v,
          bufA, bufB, bb0, bb1, acc, num_out, inv_v,
          sin, s0, s1, s2, sout, sh0, sh1):
    cid = lax.axis_index("c")
    sid = lax.axis_index("s")
    wid = sid * NC + cid
    base = wid * BPW
    qbase = wid * NCH

    ins = [
        pltpu.async_copy(c0_h.at[pl.ds(qbase, NCH)], idx0, sin),
        pltpu.async_copy(c1_h.at[pl.ds(qbase, NCH)], idx1, sin),
        pltpu.async_copy(c2_h.at[pl.ds(qbase, NCH)], idx2, sin),
        pltpu.async_copy(c3_h.at[pl.ds(qbase, NCH)], idx3, sin),
        pltpu.async_copy(b0_h.at[pl.ds(qbase, NCH)], ib0, sin),
        pltpu.async_copy(b1_h.at[pl.ds(qbase, NCH)], ib1, sin),
        pltpu.async_copy(histT_h.at[:, pl.ds(qbase, NCH), :], histT_v, sin),
        pltpu.async_copy(num_h.at[pl.ds(base, BPW)], num_v, sin),
        pltpu.async_copy(wt_h, wt_v, sin),
        pltpu.async_copy(b_h, b_v, sin),
    ]
    for cp in ins:
        cp.wait()

    def gather(table, idx, dst, sem):
        ds = [pltpu.async_copy(table.at[idx.at[q]],
                               dst.at[pl.ds(q * CH, CH), :], sem)
              for q in range(NCH)]
        return ds

    def wait_all(ds):
        for d in ds:
            d.wait()

    g0 = gather(Ec0, idx0, bufA, s0)
    g1 = gather(Ec1, idx1, bufB, s1)
    gb0 = gather(Eb0, ib0, bb0, s2)
    gb1 = gather(Eb1, ib1, bb1, s2)
    wait_all(g0)
    w0 = pltpu.async_copy(bufA, o_c0.at[pl.ds(base, BPW)], sout)
    wait_all(g1)
    w1 = pltpu.async_copy(bufB, o_c1.at[pl.ds(base, BPW)], sout)
    w0.wait()
    g2 = gather(Ec2, idx2, bufA, s0)
    w1.wait()
    g3 = gather(Ec3, idx3, bufB, s1)
    wait_all(gb0)
    wb0 = pltpu.async_copy(bb0, o_b0.at[pl.ds(base, BPW)], sout)
    wait_all(gb1)
    wb1 = pltpu.async_copy(bb1, o_b1.at[pl.ds(base, BPW)], sout)
    wait_all(g2)
    w2 = pltpu.async_copy(bufA, o_c2.at[pl.ds(base, BPW)], sout)
    wait_all(g3)
    w3 = pltpu.async_copy(bufB, o_c3.at[pl.ds(base, BPW)], sout)

    wvec = [(wt_v[k, pl.ds(0, 16)], wt_v[k, pl.ds(16, 16)]) for k in range(13)]
    bv0 = b_v[pl.ds(0, 16)]
    bv1 = b_v[pl.ds(16, 16)]

    def num_body(r, carry):
        nv = num_v[r, pl.ds(0, 16)]
        a0, a1 = bv0, bv1
        for k in range(13):
            sv = jnp.broadcast_to(nv[k], (16,))
            a0 = a0 + sv * wvec[k][0]
            a1 = a1 + sv * wvec[k][1]
        num_out[r, pl.ds(0, 16)] = a0
        num_out[r, pl.ds(16, 16)] = a1
        return carry

    lax.fori_loop(0, BPW, num_body, 0, unroll=2)
    wn = pltpu.async_copy(num_out, o_num.at[pl.ds(base, BPW)], sout)

    w2.wait()
    w3.wait()
    wn.wait()
    gbuf = (bufA, bufB, num_out)
    shs = (sh0, sh1, s0)

    def hist_gather(j, p):
        return [pltpu.async_copy(Eh.at[histT_v.at[j, q]],
                                 gbuf[p].at[pl.ds(q * CH, CH), :], shs[p])
                for q in range(NCH)]

    def mk_acc_loop(gb, first):
        def acc_body(r, carry):
            g0v = gb[r, pl.ds(0, 16)]
            g1v = gb[r, pl.ds(16, 16)]
            if first:
                acc[r, pl.ds(0, 16)] = g0v
                acc[r, pl.ds(16, 16)] = g1v
            else:
                plsc.addupdate(acc.at[r, pl.ds(0, 16)], g0v)
                plsc.addupdate(acc.at[r, pl.ds(16, 16)], g1v)
            return carry
        return acc_body

    pend = [hist_gather(0, 0), hist_gather(1, 1), hist_gather(2, 2)]
    for j in range(HL):
        p = j % 3
        wait_all(pend[p])
        lax.fori_loop(0, BPW, mk_acc_loop(gbuf[p], j == 0), 0, unroll=8)
        if j + 3 < HL:
            pend[p] = hist_gather(j + 3, p)

    def cnt_body(g, carry):
        q = g // (CH // 16)
        off = (g % (CH // 16)) * 16
        c = jnp.zeros((16,), jnp.float32)
        zi = jnp.zeros((16,), jnp.int32)
        one = jnp.full((16,), 1.0, jnp.float32)
        zf = jnp.zeros((16,), jnp.float32)
        for j in range(HL):
            iv = histT_v[j, q, pl.ds(off, 16)]
            c = c + jnp.where(iv != zi, one, zf)
        inv_v[pl.ds(g * 16, 16)] = one / jnp.maximum(c, jnp.full((16,), 1e-6, jnp.float32))
        return carry

    lax.fori_loop(0, BPW // 16, cnt_body, 0)

    def scale_body(g, carry):
        iv = inv_v[pl.ds(g * 16, 16)]
        for i in range(16):
            sv = jnp.broadcast_to(iv[i], (16,))
            r = g * 16 + i
            acc[r, pl.ds(0, 16)] = acc[r, pl.ds(0, 16)] * sv
            acc[r, pl.ds(16, 16)] = acc[r, pl.ds(16, 16)] * sv
        return carry

    lax.fori_loop(0, BPW // 16, scale_body, 0)
    wh = pltpu.async_copy(acc, o_h.at[pl.ds(base, BPW)], sout)

    for d in (wb0, wb1, wh):
        d.wait()


_encode = pl.kernel(
    _body,
    out_type=[
        jax.ShapeDtypeStruct((B, D), jnp.float32),
        jax.ShapeDtypeStruct((B, D), jnp.float32),
        jax.ShapeDtypeStruct((B, D), jnp.float32),
        jax.ShapeDtypeStruct((B, D), jnp.float32),
        jax.ShapeDtypeStruct((B, D), jnp.float32),
        jax.ShapeDtypeStruct((B, DB), jnp.float32),
        jax.ShapeDtypeStruct((B, DB), jnp.float32),
        jax.ShapeDtypeStruct((B, D), jnp.float32),
    ],
    mesh=_mesh,
    compiler_params=pltpu.CompilerParams(use_tc_tiling_on_sc=False),
    scratch_types=[
        pltpu.VMEM((NCH, CH), jnp.int32),
        pltpu.VMEM((NCH, CH), jnp.int32),
        pltpu.VMEM((NCH, CH), jnp.int32),
        pltpu.VMEM((NCH, CH), jnp.int32),
        pltpu.VMEM((NCH, CH), jnp.int32),
        pltpu.VMEM((NCH, CH), jnp.int32),
        pltpu.VMEM((HL, NCH, CH), jnp.int32),
        pltpu.VMEM((BPW, 16), jnp.float32),
        pltpu.VMEM((13, D), jnp.float32),
        pltpu.VMEM((D,), jnp.float32),
        pltpu.VMEM((BPW, D), jnp.float32),
        pltpu.VMEM((BPW, D), jnp.float32),
        pltpu.VMEM((BPW, DB), jnp.float32),
        pltpu.VMEM((BPW, DB), jnp.float32),
        pltpu.VMEM((BPW, D), jnp.float32),
        pltpu.VMEM((BPW, D), jnp.float32),
        pltpu.VMEM((BPW,), jnp.float32),
        pltpu.SemaphoreType.DMA,
        pltpu.SemaphoreType.DMA,
        pltpu.SemaphoreType.DMA,
        pltpu.SemaphoreType.DMA,
        pltpu.SemaphoreType.DMA,
        pltpu.SemaphoreType.DMA,
        pltpu.SemaphoreType.DMA,
    ],
)


def kernel(numeric, cat_0, cat_1, cat_2, cat_3, bkt_0, bkt_1, hist_items,
           W_num, b_num, E_cat_0, E_cat_1, E_cat_2, E_cat_3,
           E_bkt_0, E_bkt_1, E_hist):
    numeric = jnp.pad(numeric, ((0, 0), (0, 3)))
    hist_T = jnp.transpose(hist_items).reshape(HL, B // CH, CH)
    c0 = cat_0.reshape(B // CH, CH)
    c1 = cat_1.reshape(B // CH, CH)
    c2 = cat_2.reshape(B // CH, CH)
    c3 = cat_3.reshape(B // CH, CH)
    b0 = bkt_0.reshape(B // CH, CH)
    b1 = bkt_1.reshape(B // CH, CH)
    w_t = jnp.transpose(W_num)
    outs = _encode(numeric, c0, c1, c2, c3, b0, b1, hist_T, w_t, b_num,
                   E_cat_0, E_cat_1, E_cat_2, E_cat_3, E_bkt_0, E_bkt_1,
                   E_hist)
    return jnp.concatenate(outs, axis=-1)
